# baseline (speedup 1.0000x reference)
import math

import jax
import jax.numpy as jnp
from jax import lax
from jax.experimental import pallas as pl
from jax.experimental.pallas import tpu as pltpu
from jax.experimental.pallas import tpu_sc as plsc

N = 10000
E = 320000
IN_DIM = 128
H = 8
D = 16
EPS = 1e-09
SCALE = 1.0 / math.sqrt(D)

NC = 2
NS = 16
NW = NC * NS
EPW = E // NW
CH = 40
NCHUNK = EPW // CH
WB = 40
NWB = N // WB
WB_PER_TILE = -(-NWB // NS)


def _proj_body(x_ref, wq_ref, bq_ref, wk_ref, bk_ref, wv_ref, bv_ref,
               q_ref, k_ref, v_ref):
    xb = x_ref[...]
    q_ref[...] = jnp.dot(xb, wq_ref[...],
                         preferred_element_type=jnp.float32) + bq_ref[...]
    k_ref[...] = jnp.dot(xb, wk_ref[...],
                         preferred_element_type=jnp.float32) + bk_ref[...]
    v_ref[...] = jnp.dot(xb, wv_ref[...],
                         preferred_element_type=jnp.float32) + bv_ref[...]


def _node_proj(x, WQ, bQ, WK, bK, WV, bV):
    blk = 1000
    grid = N // blk
    wspec = pl.BlockSpec((IN_DIM, H * D), lambda i: (0, 0))
    bspec = pl.BlockSpec((1, H * D), lambda i: (0, 0))
    ospec = pl.BlockSpec((blk, H * D), lambda i: (i, 0))
    return pl.pallas_call(
        _proj_body,
        grid=(grid,),
        in_specs=[pl.BlockSpec((blk, IN_DIM), lambda i: (i, 0)),
                  wspec, bspec, wspec, bspec, wspec, bspec],
        out_specs=[ospec, ospec, ospec],
        out_shape=[jax.ShapeDtypeStruct((N, H * D), jnp.float32)] * 3,
    )(x, WQ, bQ.reshape(1, -1), WK, bK.reshape(1, -1), WV, bV.reshape(1, -1))


def _edge_proj_body(ea_ref, we_ref, be_ref, eh_ref):
    eh_ref[...] = jnp.dot(ea_ref[...], we_ref[...],
                          preferred_element_type=jnp.float32) + be_ref[...]


def _edge_proj(edge_attr, WE, bE):
    blk = 4000
    grid = E // blk
    return pl.pallas_call(
        _edge_proj_body,
        grid=(grid,),
        in_specs=[pl.BlockSpec((blk, IN_DIM), lambda i: (i, 0)),
                  pl.BlockSpec((IN_DIM, H * D), lambda i: (0, 0)),
                  pl.BlockSpec((1, H * D), lambda i: (0, 0))],
        out_specs=pl.BlockSpec((blk, H * D), lambda i: (i, 0)),
        out_shape=jax.ShapeDtypeStruct((E, H * D), jnp.float32),
    )(edge_attr, WE, bE.reshape(1, -1))


def _edge_kernel2(src_hbm, dst_hbm, qh_hbm, kh_hbm, vh_hbm, eh_hbm,
                  owv_hbm, owz_hbm,
                  src_v, dst_v, k_v, q_v, v_v, e_v, s_v, wb_v, wbz_v,
                  sem0, sem1, semv, wv_sh, wz_sh):
    cid = lax.axis_index("c")
    sid = lax.axis_index("s")
    wid = sid * NC + cid
    sems = (sem0, sem1)

    zero16 = jnp.zeros((16,), jnp.float32)

    def _zrow(r, carry):
        for cc in range(8):
            wb_v[r, pl.ds(cc * 16, 16)] = zero16
        wbz_v[r, :] = zero16
        return carry
    lax.fori_loop(0, WB, _zrow, 0)

    def _initj(j, carry):
        ci = sid + j * NS
        @pl.when(ci < NWB)
        def _init():
            r0 = pl.multiple_of(ci * WB, 8)
            pltpu.sync_copy(wb_v, wv_sh.at[pl.ds(r0, WB)])
            pltpu.sync_copy(wbz_v, wz_sh.at[pl.ds(r0, WB)])
        return carry
    lax.fori_loop(0, WB_PER_TILE, _initj, 0)
    plsc.subcore_barrier()

    lane = lax.iota(jnp.int32, 16)
    onehot = [(lane == h).astype(jnp.float32) for h in range(H)]

    ebase = wid * EPW

    def _prefetch(c, b):
        e0 = ebase + c * CH
        pltpu.sync_copy(src_hbm.at[pl.ds(e0, CH)], src_v.at[b])
        pltpu.sync_copy(dst_hbm.at[pl.ds(e0, CH)], dst_v.at[b])
        pltpu.async_copy(kh_hbm.at[src_v.at[b]], k_v.at[b], sems[b])
        pltpu.async_copy(qh_hbm.at[dst_v.at[b]], q_v.at[b], sems[b])

    def _wait(b):
        pltpu.make_async_copy(kh_hbm.at[src_v.at[b]], k_v.at[b], sems[b]).wait()
        pltpu.make_async_copy(qh_hbm.at[dst_v.at[b]], q_v.at[b], sems[b]).wait()

    def _compute(c, b):
        e0 = ebase + c * CH
        cp_v = pltpu.async_copy(vh_hbm.at[src_v.at[b]], v_v, semv)
        pltpu.sync_copy(eh_hbm.at[pl.ds(e0, CH)], e_v)
        _wait(b)
        kb, qb = k_v.at[b], q_v.at[b]

        def _dots(e, ecarry):
            srow = zero16
            for h in range(H):
                sl = pl.ds(h * 16, 16)
                dot = jnp.sum(kb[e, sl] * qb[e, sl] * e_v[e, sl]) * SCALE
                svec = jnp.exp(jnp.broadcast_to(dot, (16,)))
                srow = srow + svec * onehot[h]
            s_v[e, :] = srow
            return ecarry
        lax.fori_loop(0, CH, _dots, 0)

        cp_v.wait()

        def _scale(e, ecarry):
            srow = s_v[e, :]
            for h in range(H):
                sl = pl.ds(h * 16, 16)
                sv = jnp.take(srow, jnp.broadcast_to(h, (16,)))
                v_v[e, sl] = v_v[e, sl] * sv
            return ecarry
        lax.fori_loop(0, CH, _scale, 0)

        pltpu.sync_copy(v_v, wv_sh.at[dst_v.at[b]], add=True)
        pltpu.sync_copy(s_v, wz_sh.at[dst_v.at[b]], add=True)

    _prefetch(0, 0)

    def _pair(i, carry):
        c0 = i * 2
        _prefetch(c0 + 1, 1)
        _compute(c0, 0)
        @pl.when(c0 + 2 < NCHUNK)
        def _pf0():
            _prefetch(c0 + 2, 0)
        _compute(c0 + 1, 1)
        return carry
    lax.fori_loop(0, NCHUNK // 2, _pair, 0)

    plsc.subcore_barrier()

    def _wbj(j, carry):
        ci = sid + j * NS
        @pl.when(ci < NWB)
        def _wb():
            r0 = pl.multiple_of(ci * WB, 8)
            ro = pl.multiple_of(cid * N + r0, 8)
            pltpu.sync_copy(wv_sh.at[pl.ds(r0, WB)], wb_v)
            pltpu.sync_copy(wb_v, owv_hbm.at[pl.ds(ro, WB)])
            pltpu.sync_copy(wz_sh.at[pl.ds(r0, WB)], wbz_v)
            pltpu.sync_copy(wbz_v, owz_hbm.at[pl.ds(ro, WB)])
        return carry
    lax.fori_loop(0, WB_PER_TILE, _wbj, 0)


def _edge_stage2(src, dst, Qh, Kh, Vh, Eh):
    mesh = plsc.VectorSubcoreMesh(core_axis_name="c", subcore_axis_name="s")
    f = pl.kernel(
        _edge_kernel2,
        out_type=[jax.ShapeDtypeStruct((NC * N, H * D), jnp.float32),
                  jax.ShapeDtypeStruct((NC * N, D), jnp.float32)],
        mesh=mesh,
        compiler_params=pltpu.CompilerParams(needs_layout_passes=False,
                                             use_tc_tiling_on_sc=False),
        scratch_types=[
            pltpu.VMEM((2, CH), jnp.int32),
            pltpu.VMEM((2, CH), jnp.int32),
            pltpu.VMEM((2, CH, H * D), jnp.float32),
            pltpu.VMEM((2, CH, H * D), jnp.float32),
            pltpu.VMEM((CH, H * D), jnp.float32),
            pltpu.VMEM((CH, H * D), jnp.float32),
            pltpu.VMEM((CH, D), jnp.float32),
            pltpu.VMEM((WB, H * D), jnp.float32),
            pltpu.VMEM((WB, D), jnp.float32),
            pltpu.SemaphoreType.DMA,
            pltpu.SemaphoreType.DMA,
            pltpu.SemaphoreType.DMA,
            pltpu.VMEM_SHARED((N, H * D), jnp.float32),
            pltpu.VMEM_SHARED((N, D), jnp.float32),
        ],
    )
    return f(src, dst, Qh, Kh, Vh, Eh)


def _final_body(wv_ref, wz_ref, out_ref):
    wv = wv_ref[0] + wv_ref[1]
    wz = wz_ref[0] + wz_ref[1]
    for h in range(H):
        denom = wz[:, h:h + 1] + EPS
        out_ref[:, h * D:(h + 1) * D] = wv[:, h * D:(h + 1) * D] / denom


def _finalize(owv, owz):
    blk = 1000
    grid = N // blk
    wv2 = owv.reshape(NC, N, H * D)
    wz2 = owz.reshape(NC, N, D)
    return pl.pallas_call(
        _final_body,
        grid=(grid,),
        in_specs=[pl.BlockSpec((NC, blk, H * D), lambda i: (0, i, 0)),
                  pl.BlockSpec((NC, blk, D), lambda i: (0, i, 0))],
        out_specs=pl.BlockSpec((blk, H * D), lambda i: (i, 0)),
        out_shape=jax.ShapeDtypeStruct((N, H * D), jnp.float32),
    )(wv2, wz2)


def kernel(x, edge_attr, edge_index, WQ, bQ, WK, bK, WV, bV, WE, bE):
    Qh, Kh, Vh = _node_proj(x, WQ, bQ, WK, bK, WV, bV)
    Eh = _edge_proj(edge_attr, WE, bE)
    src = edge_index[0]
    dst = edge_index[1]
    owv, owz = _edge_stage2(src, dst, Qh, Kh, Vh, Eh)
    out = _finalize(owv, owz)
    return out.reshape(N, H, D)
